# Initial kernel scaffold; baseline (speedup 1.0000x reference)
import jax
import jax.numpy as jnp
from jax.experimental import pallas as pl


def kernel(features, memory_bank, stats_min, stats_max):
    raise NotImplementedError("write your pallas kernel here")


# speedup vs baseline: 6.1434x; 6.1434x over previous
import functools

import jax
import jax.numpy as jnp
from jax.experimental import pallas as pl
from jax.experimental.pallas import tpu as pltpu

_Q = 1024
_D = 16
_K = 100000
_KB = 2048
_K_PAD = 100352
_NBLK = _K_PAD // _KB


def _nn_kernel(feat_ref, mb_ref, stats_ref, out_ref):
    k = pl.program_id(0)

    feat = feat_ref[...]
    mb = mb_ref[...]

    b_sq = jnp.sum(mb * mb, axis=1)
    col = k * _KB + jax.lax.iota(jnp.int32, _KB)
    b_sq = jnp.where(col < _K, b_sq, jnp.inf)

    dots = jax.lax.dot_general(
        feat, mb,
        dimension_numbers=(((1,), (1,)), ((), ())),
        preferred_element_type=jnp.float32,
    )
    sq = b_sq[None, :] - 2.0 * dots
    part = jnp.min(sq, axis=1)[:, None]

    @pl.when(k == 0)
    def _init():
        out_ref[...] = part

    @pl.when(k > 0)
    def _acc():
        out_ref[...] = jnp.minimum(out_ref[...], part)

    @pl.when(k == _NBLK - 1)
    def _finish():
        a_sq = jnp.sum(feat * feat, axis=1)[:, None]
        sq_min = jnp.maximum(out_ref[...] + a_sq, 1e-12)
        dist = jnp.sqrt(sq_min)
        s_min = stats_ref[0]
        s_max = stats_ref[1]
        out_ref[...] = (dist - s_min) / (s_max - s_min)


@functools.partial(jax.jit, static_argnames=())
def _run(features, memory_bank, stats):
    mb_padded = jnp.pad(memory_bank, ((0, _K_PAD - _K), (0, 0)))
    out = pl.pallas_call(
        _nn_kernel,
        grid=(_NBLK,),
        in_specs=[
            pl.BlockSpec((_Q, _D), lambda k: (0, 0)),
            pl.BlockSpec((_KB, _D), lambda k: (k, 0)),
            pl.BlockSpec(memory_space=pltpu.SMEM),
        ],
        out_specs=pl.BlockSpec((_Q, 1), lambda k: (0, 0)),
        out_shape=jax.ShapeDtypeStruct((_Q, 1), jnp.float32),
    )(features, mb_padded, stats)
    return out[:, 0]


def kernel(features, memory_bank, stats_min, stats_max):
    stats = jnp.stack([jnp.asarray(stats_min, jnp.float32),
                       jnp.asarray(stats_max, jnp.float32)])
    return _run(features, memory_bank, stats)


# speedup vs baseline: 10.4696x; 1.7042x over previous
import functools

import jax
import jax.numpy as jnp
from jax.experimental import pallas as pl
from jax.experimental.pallas import tpu as pltpu

_Q = 1024
_D = 16
_DA = 32
_K = 100000
_KB = 2048
_K_PAD = 100352
_NBLK = _K_PAD // _KB
_LANES = 128


def _nn_kernel(feat_ref, mbt_ref, stats_ref, out_ref, afeat_ref, amb_ref,
               macc_ref):
    k = pl.program_id(0)

    @pl.when(k == 0)
    def _init():
        afeat_ref[...] = jnp.zeros((_Q, _DA), jnp.float32)
        afeat_ref[:, 0:_D] = -2.0 * feat_ref[...]
        afeat_ref[:, _D:_D + 1] = jnp.ones((_Q, 1), jnp.float32)
        amb_ref[...] = jnp.zeros((_DA, _KB), jnp.float32)

    mbt = mbt_ref[...]
    b_sq = jnp.sum(mbt * mbt, axis=0, keepdims=True)
    col = k * _KB + jax.lax.iota(jnp.int32, _KB)[None, :]
    b_sq = jnp.where(col < _K, b_sq, jnp.inf)

    amb_ref[0:_D, :] = mbt
    amb_ref[_D:_D + 1, :] = b_sq

    sq = jax.lax.dot_general(
        afeat_ref[...], amb_ref[...],
        dimension_numbers=(((1,), (0,)), ((), ())),
        preferred_element_type=jnp.float32,
    )

    m = sq[:, 0:_LANES]
    for i in range(1, _KB // _LANES):
        m = jnp.minimum(m, sq[:, i * _LANES:(i + 1) * _LANES])

    @pl.when(k == 0)
    def _first():
        macc_ref[...] = m

    @pl.when(k > 0)
    def _acc():
        macc_ref[...] = jnp.minimum(macc_ref[...], m)

    @pl.when(k == _NBLK - 1)
    def _finish():
        feat = feat_ref[...]
        a_sq = jnp.sum(feat * feat, axis=1)[:, None]
        row_min = jnp.min(macc_ref[...], axis=1)[:, None]
        sq_min = jnp.maximum(row_min + a_sq, 1e-12)
        dist = jnp.sqrt(sq_min)
        s_min = stats_ref[0]
        s_max = stats_ref[1]
        out_ref[...] = (dist - s_min) / (s_max - s_min)


@functools.partial(jax.jit, static_argnames=())
def _run(features, memory_bank, stats):
    mbt = jnp.pad(memory_bank, ((0, _K_PAD - _K), (0, 0))).T
    out = pl.pallas_call(
        _nn_kernel,
        grid=(_NBLK,),
        in_specs=[
            pl.BlockSpec((_Q, _D), lambda k: (0, 0)),
            pl.BlockSpec((_D, _KB), lambda k: (0, k)),
            pl.BlockSpec(memory_space=pltpu.SMEM),
        ],
        out_specs=pl.BlockSpec((_Q, 1), lambda k: (0, 0)),
        out_shape=jax.ShapeDtypeStruct((_Q, 1), jnp.float32),
        scratch_shapes=[
            pltpu.VMEM((_Q, _DA), jnp.float32),
            pltpu.VMEM((_DA, _KB), jnp.float32),
            pltpu.VMEM((_Q, _LANES), jnp.float32),
        ],
    )(features, mbt, stats)
    return out[:, 0]


def kernel(features, memory_bank, stats_min, stats_max):
    stats = jnp.stack([jnp.asarray(stats_min, jnp.float32),
                       jnp.asarray(stats_max, jnp.float32)])
    return _run(features, memory_bank, stats)


# speedup vs baseline: 10.4906x; 1.0020x over previous
import functools

import jax
import jax.numpy as jnp
from jax.experimental import pallas as pl
from jax.experimental.pallas import tpu as pltpu

_Q = 1024
_D = 16
_DA = 32
_K = 100000
_KB = 2048
_K_PAD = 100352
_NBLK = _K_PAD // _KB
_LANES = 128


def _nn_kernel(feat_ref, mbt_ref, stats_ref, out_ref, afeat_ref, amb_ref,
               macc_ref):
    k = pl.program_id(0)

    @pl.when(k == 0)
    def _init():
        afeat_ref[...] = jnp.zeros((_Q, _DA), jnp.float32)
        afeat_ref[:, 0:_D] = -2.0 * feat_ref[...]
        afeat_ref[:, _D:_D + 1] = jnp.ones((_Q, 1), jnp.float32)
        amb_ref[...] = jnp.zeros((_DA, _KB), jnp.float32)

    mbt = mbt_ref[...]
    b_sq = jnp.sum(mbt * mbt, axis=0, keepdims=True)
    col = k * _KB + jax.lax.iota(jnp.int32, _KB)[None, :]
    b_sq = jnp.where(col < _K, b_sq, jnp.inf)

    amb_ref[0:_D, :] = mbt
    amb_ref[_D:_D + 1, :] = b_sq

    afeat = afeat_ref[...]

    def _chunk(i):
        return jax.lax.dot_general(
            afeat, amb_ref[:, i * _LANES:(i + 1) * _LANES],
            dimension_numbers=(((1,), (0,)), ((), ())),
            preferred_element_type=jnp.float32,
        )

    m = _chunk(0)
    for i in range(1, _KB // _LANES):
        m = jnp.minimum(m, _chunk(i))

    @pl.when(k == 0)
    def _first():
        macc_ref[...] = m

    @pl.when(k > 0)
    def _acc():
        macc_ref[...] = jnp.minimum(macc_ref[...], m)

    @pl.when(k == _NBLK - 1)
    def _finish():
        feat = feat_ref[...]
        a_sq = jnp.sum(feat * feat, axis=1)[:, None]
        row_min = jnp.min(macc_ref[...], axis=1)[:, None]
        sq_min = jnp.maximum(row_min + a_sq, 1e-12)
        dist = jnp.sqrt(sq_min)
        s_min = stats_ref[0]
        s_max = stats_ref[1]
        out_ref[...] = (dist - s_min) / (s_max - s_min)


@functools.partial(jax.jit, static_argnames=())
def _run(features, memory_bank, stats):
    mbt = jnp.pad(memory_bank, ((0, _K_PAD - _K), (0, 0))).T
    out = pl.pallas_call(
        _nn_kernel,
        grid=(_NBLK,),
        in_specs=[
            pl.BlockSpec((_Q, _D), lambda k: (0, 0)),
            pl.BlockSpec((_D, _KB), lambda k: (0, k)),
            pl.BlockSpec(memory_space=pltpu.SMEM),
        ],
        out_specs=pl.BlockSpec((_Q, 1), lambda k: (0, 0)),
        out_shape=jax.ShapeDtypeStruct((_Q, 1), jnp.float32),
        scratch_shapes=[
            pltpu.VMEM((_Q, _DA), jnp.float32),
            pltpu.VMEM((_DA, _KB), jnp.float32),
            pltpu.VMEM((_Q, _LANES), jnp.float32),
        ],
    )(features, mbt, stats)
    return out[:, 0]


def kernel(features, memory_bank, stats_min, stats_max):
    stats = jnp.stack([jnp.asarray(stats_min, jnp.float32),
                       jnp.asarray(stats_max, jnp.float32)])
    return _run(features, memory_bank, stats)


# speedup vs baseline: 11.1657x; 1.0644x over previous
import functools

import jax
import jax.numpy as jnp
from jax.experimental import pallas as pl
from jax.experimental.pallas import tpu as pltpu

_Q = 1024
_D = 16
_DA = 32
_K = 100000
_KB = 4096
_K_PAD = 102400
_NBLK = _K_PAD // _KB
_LANES = 128


def _nn_kernel(feat_ref, mbt_ref, stats_ref, out_ref, afeat_ref, amb_ref,
               macc_ref):
    k = pl.program_id(0)

    @pl.when(k == 0)
    def _init():
        afeat_ref[...] = jnp.zeros((_Q, _DA), jnp.float32)
        afeat_ref[:, 0:_D] = -2.0 * feat_ref[...]
        afeat_ref[:, _D:_D + 1] = jnp.ones((_Q, 1), jnp.float32)
        amb_ref[...] = jnp.zeros((_DA, _KB), jnp.float32)

    mbt = mbt_ref[...]
    b_sq = jnp.sum(mbt * mbt, axis=0, keepdims=True)
    col = k * _KB + jax.lax.iota(jnp.int32, _KB)[None, :]
    b_sq = jnp.where(col < _K, b_sq, jnp.inf)

    amb_ref[0:_D, :] = mbt
    amb_ref[_D:_D + 1, :] = b_sq

    afeat = afeat_ref[...]

    def _chunk(i):
        return jax.lax.dot_general(
            afeat, amb_ref[:, i * _LANES:(i + 1) * _LANES],
            dimension_numbers=(((1,), (0,)), ((), ())),
            preferred_element_type=jnp.float32,
        )

    m = _chunk(0)
    for i in range(1, _KB // _LANES):
        m = jnp.minimum(m, _chunk(i))

    @pl.when(k == 0)
    def _first():
        macc_ref[...] = m

    @pl.when(k > 0)
    def _acc():
        macc_ref[...] = jnp.minimum(macc_ref[...], m)

    @pl.when(k == _NBLK - 1)
    def _finish():
        feat = feat_ref[...]
        a_sq = jnp.sum(feat * feat, axis=1)[:, None]
        row_min = jnp.min(macc_ref[...], axis=1)[:, None]
        sq_min = jnp.maximum(row_min + a_sq, 1e-12)
        dist = jnp.sqrt(sq_min)
        s_min = stats_ref[0]
        s_max = stats_ref[1]
        out_ref[...] = (dist - s_min) / (s_max - s_min)


@functools.partial(jax.jit, static_argnames=())
def _run(features, memory_bank, stats):
    mbt = jnp.pad(memory_bank, ((0, _K_PAD - _K), (0, 0))).T
    out = pl.pallas_call(
        _nn_kernel,
        grid=(_NBLK,),
        in_specs=[
            pl.BlockSpec((_Q, _D), lambda k: (0, 0)),
            pl.BlockSpec((_D, _KB), lambda k: (0, k)),
            pl.BlockSpec(memory_space=pltpu.SMEM),
        ],
        out_specs=pl.BlockSpec((_Q, 1), lambda k: (0, 0)),
        out_shape=jax.ShapeDtypeStruct((_Q, 1), jnp.float32),
        scratch_shapes=[
            pltpu.VMEM((_Q, _DA), jnp.float32),
            pltpu.VMEM((_DA, _KB), jnp.float32),
            pltpu.VMEM((_Q, _LANES), jnp.float32),
        ],
    )(features, mbt, stats)
    return out[:, 0]


def kernel(features, memory_bank, stats_min, stats_max):
    stats = jnp.stack([jnp.asarray(stats_min, jnp.float32),
                       jnp.asarray(stats_max, jnp.float32)])
    return _run(features, memory_bank, stats)
